# baseline (speedup 1.0000x reference)
import functools

import jax
import jax.numpy as jnp
from jax.experimental import pallas as pl
from jax.experimental.pallas import tpu as pltpu

_HOP = 1920
_D = 512
_K = 2048
_Q = 8
_TILE = 512


def _rvq_body(frames_ref, w_ref, b_ref, cb_ref, codes_ref,
              res_ref, cbs_ref, cn_ref, idxv_ref, idxs_ref, qnt_ref, sem):
    q = pl.program_id(0)
    i = pl.program_id(1)
    rows = pl.ds(i * _TILE, _TILE)

    @pl.when(q == 0)
    def _encode():
        lat = jnp.dot(frames_ref[...].astype(jnp.bfloat16),
                      w_ref[...].astype(jnp.bfloat16),
                      preferred_element_type=jnp.float32)
        res_ref[rows, :] = lat + b_ref[...]

    @pl.when(i == 0)
    def _prep_codebook():
        cb = cb_ref[0]
        cbs_ref[...] = cb.astype(jnp.bfloat16)
        cn_ref[...] = jnp.sum(cb * cb, axis=1)[None, :]

    r = res_ref[rows, :]
    rnorm = jnp.sum(r * r, axis=1, keepdims=True)
    scores = jax.lax.dot_general(
        r.astype(jnp.bfloat16), cbs_ref[...], (((1,), (1,)), ((), ())),
        preferred_element_type=jnp.float32)
    d = rnorm - 2.0 * scores + cn_ref[...]

    idx = jnp.argmin(d, axis=1).astype(jnp.int32)
    codes_ref[0, 0, :] = idx

    @pl.when(q < _Q - 1)
    def _gather_update():
        idxv_ref[...] = idx[None, :]
        cp = pltpu.make_async_copy(idxv_ref, idxs_ref, sem)
        cp.start()
        cp.wait()

        def _copy_row(j, _):
            k = idxs_ref[0, j]
            qnt_ref[pl.ds(j, 1), :] = cb_ref[0, pl.ds(k, 1), :]
            return 0

        jax.lax.fori_loop(0, _TILE, _copy_row, 0, unroll=32)
        res_ref[rows, :] = r - qnt_ref[...]


@functools.partial(jax.jit, static_argnames=())
def kernel(audio_input, W_enc, b_enc, codebooks):
    B = audio_input.shape[0]
    x = audio_input.reshape(B, -1)
    T = x.shape[1] // _HOP
    rows = B * T
    frames = x[:, : T * _HOP].reshape(rows, _HOP)
    n_tiles = (rows + _TILE - 1) // _TILE
    padded = n_tiles * _TILE
    if padded != rows:
        frames = jnp.concatenate(
            [frames, jnp.zeros((padded - rows, _HOP), jnp.float32)], axis=0)

    codes = pl.pallas_call(
        _rvq_body,
        grid=(_Q, n_tiles),
        in_specs=[
            pl.BlockSpec((_TILE, _HOP), lambda q, i: (i, 0)),
            pl.BlockSpec((_HOP, _D), lambda q, i: (0, 0)),
            pl.BlockSpec((1, _D), lambda q, i: (0, 0)),
            pl.BlockSpec((1, _K, _D), lambda q, i: (q, 0, 0)),
        ],
        out_specs=pl.BlockSpec(
            (1, 1, _TILE), lambda q, i, nt=n_tiles: (q * nt + i, 0, 0)),
        out_shape=jax.ShapeDtypeStruct((_Q * n_tiles, 1, _TILE), jnp.int32),
        scratch_shapes=[pltpu.VMEM((padded, _D), jnp.float32),
                        pltpu.VMEM((_K, _D), jnp.bfloat16),
                        pltpu.VMEM((1, _K), jnp.float32),
                        pltpu.VMEM((1, _TILE), jnp.int32),
                        pltpu.SMEM((1, _TILE), jnp.int32),
                        pltpu.VMEM((_TILE, _D), jnp.float32),
                        pltpu.SemaphoreType.DMA],
    )(frames, W_enc, b_enc.reshape(1, _D), codebooks)

    codes = codes.reshape(_Q, padded)[:, :rows]
    codes = codes.reshape(_Q, B, T).transpose(1, 0, 2)
    return codes.astype(jnp.int32)
